# Initial kernel scaffold; baseline (speedup 1.0000x reference)
import jax
import jax.numpy as jnp
from jax.experimental import pallas as pl


def kernel(positive_edges, negative_edges, g, x_u, x_v, r):
    raise NotImplementedError("write your pallas kernel here")


# speedup vs baseline: 2.2605x; 2.2605x over previous
import functools

import jax
import jax.numpy as jnp
from jax import lax
from jax.experimental import pallas as pl
from jax.experimental.pallas import tpu as pltpu
from jax.experimental.pallas import tpu_sc as plsc

D = 128
LANES = 16
VPR = D // LANES

_info = plsc.get_sparse_core_info()
NC, NS = _info.num_cores, _info.num_subcores
NW = NC * NS

CHUNK = 80
UNROLL = 4


def _make_score_kernel(num_edges):
    assert num_edges % NW == 0
    per_w = num_edges // NW
    assert per_w % CHUNK == 0
    n_chunks = per_w // CHUNK

    mesh = plsc.VectorSubcoreMesh(core_axis_name="c", subcore_axis_name="s")

    @functools.partial(
        pl.kernel,
        mesh=mesh,
        compiler_params=pltpu.CompilerParams(needs_layout_passes=False),
        out_type=jax.ShapeDtypeStruct((num_edges,), jnp.float32),
        scratch_types=[
            pltpu.VMEM((CHUNK,), jnp.int32),
            pltpu.VMEM((CHUNK,), jnp.int32),
            pltpu.VMEM((CHUNK, D), jnp.float32),
            pltpu.VMEM((CHUNK, D), jnp.float32),
            pltpu.VMEM((CHUNK,), jnp.float32),
            pltpu.VMEM((D,), jnp.float32),
            pltpu.SemaphoreType.DMA,
        ],
    )
    def score_kernel(srcs_hbm, dsts_hbm, xu_hbm, xv_hbm, r_hbm, out_hbm,
                     src_idx, dst_idx, u_rows, v_rows, scores, r_v, sem):
        wid = lax.axis_index("s") * NC + lax.axis_index("c")
        base = wid * per_w

        pltpu.sync_copy(r_hbm, r_v)
        r_regs = [r_v[pl.ds(LANES * j, LANES)] for j in range(VPR)]

        def chunk_body(c, carry):
            off = base + c * CHUNK
            pltpu.sync_copy(srcs_hbm.at[pl.ds(off, CHUNK)], src_idx)
            pltpu.sync_copy(dsts_hbm.at[pl.ds(off, CHUNK)], dst_idx)
            cp_u = pltpu.async_copy(xu_hbm.at[src_idx], u_rows, sem)
            cp_v = pltpu.async_copy(xv_hbm.at[dst_idx], v_rows, sem)
            cp_u.wait()
            cp_v.wait()

            lane = lax.broadcasted_iota(jnp.int32, (LANES,), 0)

            def group_body(gb, carry2):
                acc = jnp.zeros((LANES,), jnp.float32)
                for k in range(LANES):
                    e = gb * LANES + k
                    parts = []
                    for j in range(VPR):
                        u = u_rows[e, pl.ds(LANES * j, LANES)]
                        v = v_rows[e, pl.ds(LANES * j, LANES)]
                        parts.append(u * v * r_regs[j])
                    while len(parts) > 1:
                        parts = [a + b for a, b in
                                 zip(parts[::2], parts[1::2])]
                    acc = jnp.where(lane == k, jnp.sum(parts[0]), acc)
                scores[pl.ds(gb * LANES, LANES)] = acc
                return carry2

            lax.fori_loop(0, CHUNK // LANES, group_body, 0)
            pltpu.sync_copy(scores, out_hbm.at[pl.ds(off, CHUNK)])
            return carry

        lax.fori_loop(0, n_chunks, chunk_body, 0)

    return score_kernel


def kernel(positive_edges, negative_edges, g, x_u, x_v, r):
    e = positive_edges.shape[0]
    srcs = jnp.concatenate([positive_edges[:, 0], negative_edges[:, 0]])
    dsts = jnp.concatenate([positive_edges[:, 1], negative_edges[:, 1]])
    scores = _make_score_kernel(2 * e)(srcs, dsts, x_u, x_v, r)
    return (scores[:e], scores[e:])


# speedup vs baseline: 7.0338x; 3.1116x over previous
import functools

import jax
import jax.numpy as jnp
from jax import lax
from jax.experimental import pallas as pl
from jax.experimental.pallas import tpu as pltpu
from jax.experimental.pallas import tpu_sc as plsc

D = 128
LANES = 16
VPR = D // LANES
PSTRIDE = LANES + 1

_info = plsc.get_sparse_core_info()
NC, NS = _info.num_cores, _info.num_subcores
NW = NC * NS

CHUNK = 80
GROUPS = CHUNK // LANES


def _make_score_kernel(num_edges):
    assert num_edges % NW == 0
    per_w = num_edges // NW
    assert per_w % (2 * CHUNK) == 0
    n_chunks = per_w // CHUNK

    mesh = plsc.VectorSubcoreMesh(core_axis_name="c", subcore_axis_name="s")

    @functools.partial(
        pl.kernel,
        mesh=mesh,
        compiler_params=pltpu.CompilerParams(needs_layout_passes=False),
        out_type=jax.ShapeDtypeStruct((num_edges,), jnp.float32),
        scratch_types=[
            pltpu.VMEM((2, CHUNK), jnp.int32),
            pltpu.VMEM((2, CHUNK), jnp.int32),
            pltpu.VMEM((2, CHUNK, D), jnp.float32),
            pltpu.VMEM((2, CHUNK, D), jnp.float32),
            pltpu.VMEM((2, CHUNK), jnp.float32),
            pltpu.VMEM((LANES * PSTRIDE,), jnp.float32),
            pltpu.VMEM((D,), jnp.float32),
            pltpu.SemaphoreType.DMA((2,)),
            pltpu.SemaphoreType.DMA((2,)),
            pltpu.SemaphoreType.DMA((2,)),
        ],
    )
    def score_kernel(srcs_hbm, dsts_hbm, xu_hbm, xv_hbm, r_hbm, out_hbm,
                     src_idx, dst_idx, u_rows, v_rows, scores, pmat, r_v,
                     sem_rows, sem_idx, sem_out):
        wid = lax.axis_index("s") * NC + lax.axis_index("c")
        base = wid * per_w

        pltpu.sync_copy(r_hbm, r_v)
        r_regs = [r_v[pl.ds(LANES * j, LANES)] for j in range(VPR)]
        lane = lax.broadcasted_iota(jnp.int32, (LANES,), 0)
        col_base = lane * PSTRIDE

        def start_idx(c, b):
            off = base + c * CHUNK
            pltpu.async_copy(srcs_hbm.at[pl.ds(off, CHUNK)],
                             src_idx.at[b], sem_idx.at[b])
            pltpu.async_copy(dsts_hbm.at[pl.ds(off, CHUNK)],
                             dst_idx.at[b], sem_idx.at[b])

        def wait_idx(b):
            pltpu.make_async_copy(srcs_hbm.at[pl.ds(0, CHUNK)],
                                  src_idx.at[b], sem_idx.at[b]).wait()
            pltpu.make_async_copy(srcs_hbm.at[pl.ds(0, CHUNK)],
                                  dst_idx.at[b], sem_idx.at[b]).wait()

        def start_rows(b):
            pltpu.async_copy(xu_hbm.at[src_idx.at[b]],
                             u_rows.at[b], sem_rows.at[b])
            pltpu.async_copy(xv_hbm.at[dst_idx.at[b]],
                             v_rows.at[b], sem_rows.at[b])

        def wait_rows(b):
            pltpu.make_async_copy(xu_hbm.at[src_idx.at[b]],
                                  u_rows.at[b], sem_rows.at[b]).wait()
            pltpu.make_async_copy(xv_hbm.at[dst_idx.at[b]],
                                  v_rows.at[b], sem_rows.at[b]).wait()

        def compute(c, b):
            def group_body(gb, carry2):
                for k in range(LANES):
                    e = gb * LANES + k
                    parts = []
                    for j in range(VPR):
                        u = u_rows[b, e, pl.ds(LANES * j, LANES)]
                        v = v_rows[b, e, pl.ds(LANES * j, LANES)]
                        parts.append(u * v * r_regs[j])
                    while len(parts) > 1:
                        parts = [a + bb for a, bb in
                                 zip(parts[::2], parts[1::2])]
                    plsc.store_scatter(pmat, [col_base + k], parts[0])
                accs = [plsc.load_gather(pmat, [lane + l * PSTRIDE])
                        for l in range(4)]
                for l in range(4, LANES):
                    accs[l % 4] = accs[l % 4] + plsc.load_gather(
                        pmat, [lane + l * PSTRIDE])
                acc = (accs[0] + accs[1]) + (accs[2] + accs[3])
                scores[b, pl.ds(gb * LANES, LANES)] = acc
                return carry2

            lax.fori_loop(0, GROUPS, group_body, 0)
            pltpu.async_copy(scores.at[b],
                             out_hbm.at[pl.ds(base + c * CHUNK, CHUNK)],
                             sem_out.at[b])

        def wait_out(c, b):
            pltpu.make_async_copy(scores.at[b],
                                  out_hbm.at[pl.ds(0, CHUNK)],
                                  sem_out.at[b]).wait()

        start_idx(0, 0)
        start_idx(1, 1)
        wait_idx(0)
        start_rows(0)

        def loop_body(i, carry):
            for b in (0, 1):
                c = 2 * i + b
                wait_rows(b)

                @pl.when(c + 2 < n_chunks)
                def _():
                    start_idx(c + 2, b)

                @pl.when(c + 1 < n_chunks)
                def _():
                    wait_idx(1 - b)
                    start_rows(1 - b)

                @pl.when(c >= 2)
                def _():
                    wait_out(c, b)

                compute(c, b)
            return carry

        lax.fori_loop(0, n_chunks // 2, loop_body, 0)
        wait_out(n_chunks - 2, 0)
        wait_out(n_chunks - 1, 1)

    return score_kernel


def kernel(positive_edges, negative_edges, g, x_u, x_v, r):
    e = positive_edges.shape[0]
    srcs = jnp.concatenate([positive_edges[:, 0], negative_edges[:, 0]])
    dsts = jnp.concatenate([positive_edges[:, 1], negative_edges[:, 1]])
    scores = _make_score_kernel(2 * e)(srcs, dsts, x_u, x_v, r)
    return (scores[:e], scores[e:])


# speedup vs baseline: 7.1416x; 1.0153x over previous
import functools

import jax
import jax.numpy as jnp
from jax import lax
from jax.experimental import pallas as pl
from jax.experimental.pallas import tpu as pltpu
from jax.experimental.pallas import tpu_sc as plsc

D = 128
DW = D // 2
LANES = 16
PSTRIDE = LANES + 1

_info = plsc.get_sparse_core_info()
NC, NS = _info.num_cores, _info.num_subcores
NW = NC * NS

CHUNK = 80
GROUPS = CHUNK // LANES


def _make_score_kernel(num_edges):
    assert num_edges % NW == 0
    per_w = num_edges // NW
    assert per_w % (2 * CHUNK) == 0
    n_chunks = per_w // CHUNK

    mesh = plsc.VectorSubcoreMesh(core_axis_name="c", subcore_axis_name="s")

    @functools.partial(
        pl.kernel,
        mesh=mesh,
        compiler_params=pltpu.CompilerParams(
            needs_layout_passes=False, use_tc_tiling_on_sc=False),
        out_type=jax.ShapeDtypeStruct((num_edges,), jnp.float32),
        scratch_types=[
            pltpu.VMEM((2, CHUNK), jnp.int32),
            pltpu.VMEM((2, CHUNK), jnp.int32),
            pltpu.VMEM((2, CHUNK, DW), jnp.int32),
            pltpu.VMEM((2, CHUNK, DW), jnp.int32),
            pltpu.VMEM((2, CHUNK), jnp.float32),
            pltpu.VMEM((LANES * PSTRIDE,), jnp.float32),
            pltpu.VMEM((DW,), jnp.int32),
            pltpu.SemaphoreType.DMA((2,)),
            pltpu.SemaphoreType.DMA((2,)),
            pltpu.SemaphoreType.DMA((2,)),
        ],
    )
    def score_kernel(srcs_hbm, dsts_hbm, xu_hbm, xv_hbm, r_hbm, out_hbm,
                     src_idx, dst_idx, u_rows, v_rows, scores, pmat, r_v,
                     sem_rows, sem_idx, sem_out):
        wid = lax.axis_index("s") * NC + lax.axis_index("c")
        base = wid * per_w

        pltpu.sync_copy(r_hbm, r_v)
        r_regs = []
        for j in range(DW // LANES):
            rbf = plsc.bitcast(r_v[pl.ds(LANES * j, LANES)], jnp.bfloat16)
            r_regs.append(plsc.unpack(
                rbf, format=plsc.PackFormat.INTERLEAVED))
        lane = lax.broadcasted_iota(jnp.int32, (LANES,), 0)
        col_base = lane * PSTRIDE

        def start_idx(c, b):
            off = base + c * CHUNK
            pltpu.async_copy(srcs_hbm.at[pl.ds(off, CHUNK)],
                             src_idx.at[b], sem_idx.at[b])
            pltpu.async_copy(dsts_hbm.at[pl.ds(off, CHUNK)],
                             dst_idx.at[b], sem_idx.at[b])

        def wait_idx(b):
            pltpu.make_async_copy(srcs_hbm.at[pl.ds(0, CHUNK)],
                                  src_idx.at[b], sem_idx.at[b]).wait()
            pltpu.make_async_copy(srcs_hbm.at[pl.ds(0, CHUNK)],
                                  dst_idx.at[b], sem_idx.at[b]).wait()

        def start_rows(b):
            pltpu.async_copy(xu_hbm.at[src_idx.at[b]],
                             u_rows.at[b], sem_rows.at[b])
            pltpu.async_copy(xv_hbm.at[dst_idx.at[b]],
                             v_rows.at[b], sem_rows.at[b])

        def wait_rows(b):
            pltpu.make_async_copy(xu_hbm.at[src_idx.at[b]],
                                  u_rows.at[b], sem_rows.at[b]).wait()
            pltpu.make_async_copy(xv_hbm.at[dst_idx.at[b]],
                                  v_rows.at[b], sem_rows.at[b]).wait()

        def compute(c, b):
            def group_body(gb, carry2):
                for k in range(LANES):
                    e = gb * LANES + k
                    parts = []
                    for j in range(DW // LANES):
                        u = plsc.bitcast(
                            u_rows[b, e, pl.ds(LANES * j, LANES)],
                            jnp.bfloat16)
                        v = plsc.bitcast(
                            v_rows[b, e, pl.ds(LANES * j, LANES)],
                            jnp.bfloat16)
                        wa, wb = plsc.unpack(
                            u * v, format=plsc.PackFormat.INTERLEAVED)
                        ra, rb = r_regs[j]
                        parts.append(wa * ra)
                        parts.append(wb * rb)
                    while len(parts) > 1:
                        parts = [a + bb for a, bb in
                                 zip(parts[::2], parts[1::2])]
                    plsc.store_scatter(pmat, [col_base + k], parts[0])
                accs = [plsc.load_gather(pmat, [lane + l * PSTRIDE])
                        for l in range(4)]
                for l in range(4, LANES):
                    accs[l % 4] = accs[l % 4] + plsc.load_gather(
                        pmat, [lane + l * PSTRIDE])
                acc = (accs[0] + accs[1]) + (accs[2] + accs[3])
                scores[b, pl.ds(gb * LANES, LANES)] = acc
                return carry2

            lax.fori_loop(0, GROUPS, group_body, 0)
            pltpu.async_copy(scores.at[b],
                             out_hbm.at[pl.ds(base + c * CHUNK, CHUNK)],
                             sem_out.at[b])

        def wait_out(c, b):
            pltpu.make_async_copy(scores.at[b],
                                  out_hbm.at[pl.ds(0, CHUNK)],
                                  sem_out.at[b]).wait()

        start_idx(0, 0)
        start_idx(1, 1)
        wait_idx(0)
        start_rows(0)

        def loop_body(i, carry):
            for b in (0, 1):
                c = 2 * i + b
                wait_rows(b)

                @pl.when(c + 2 < n_chunks)
                def _():
                    start_idx(c + 2, b)

                @pl.when(c + 1 < n_chunks)
                def _():
                    wait_idx(1 - b)
                    start_rows(1 - b)

                @pl.when(c >= 2)
                def _():
                    wait_out(c, b)

                compute(c, b)
            return carry

        lax.fori_loop(0, n_chunks // 2, loop_body, 0)
        wait_out(n_chunks - 2, 0)
        wait_out(n_chunks - 1, 1)

    return score_kernel


def kernel(positive_edges, negative_edges, g, x_u, x_v, r):
    e = positive_edges.shape[0]
    n = x_u.shape[0]
    srcs = jnp.concatenate([positive_edges[:, 0], negative_edges[:, 0]])
    dsts = jnp.concatenate([positive_edges[:, 1], negative_edges[:, 1]])

    def to_words(t):
        t16 = t.astype(jnp.bfloat16)
        return lax.bitcast_convert_type(
            t16.reshape(*t16.shape[:-1], DW, 2), jnp.int32)

    scores = _make_score_kernel(2 * e)(
        srcs, dsts, to_words(x_u), to_words(x_v), to_words(r))
    return (scores[:e], scores[e:])
